# baseline (speedup 1.0000x reference)
import functools

import jax
import jax.numpy as jnp
from jax import lax
from jax.experimental import pallas as pl

N_ENT = 10000
N_ITEM = 5000
N_USER = 5000
D = 256
H = 128
E = 160000
NR = 9
NS = 50

_f32 = jnp.float32


def _tanh_linear_body(x_ref, w_ref, b_ref, o_ref):
    x = x_ref[...]
    y = lax.dot_general(x, w_ref[...], (((1,), (1,)), ((), ())),
                        preferred_element_type=_f32)
    o_ref[...] = jnp.tanh(y + b_ref[...])


def _tc_tanh_linear(x, w, b):
    m = x.shape[0]
    bm = 1000
    return pl.pallas_call(
        _tanh_linear_body,
        grid=(m // bm,),
        in_specs=[
            pl.BlockSpec((bm, D), lambda i: (i, 0)),
            pl.BlockSpec((D, D), lambda i: (0, 0)),
            pl.BlockSpec((1, D), lambda i: (0, 0)),
        ],
        out_specs=pl.BlockSpec((bm, D), lambda i: (i, 0)),
        out_shape=jax.ShapeDtypeStruct((m, D), _f32),
    )(x, w, b.reshape(1, D))


def _matmul_nt_body(a_ref, b_ref, o_ref):
    o_ref[...] = lax.dot_general(a_ref[...], b_ref[...],
                                 (((1,), (1,)), ((), ())),
                                 preferred_element_type=_f32)


def _tc_matmul_nt(a, b):
    m, n = a.shape[0], b.shape[0]
    bm = 1000
    return pl.pallas_call(
        _matmul_nt_body,
        grid=(m // bm,),
        in_specs=[
            pl.BlockSpec((bm, D), lambda i: (i, 0)),
            pl.BlockSpec((n, D), lambda i: (0, 0)),
        ],
        out_specs=pl.BlockSpec((bm, n), lambda i: (i, 0)),
        out_shape=jax.ShapeDtypeStruct((m, n), _f32),
    )(a, b)


def _entw_body(ent_ref, w_ref, lo_ref, hi_ref):
    r = pl.program_id(0)
    wv = w_ref[pl.ds(r, 1), :]
    prod = ent_ref[...] * wv
    lo_ref[0] = prod[:, :H]
    hi_ref[0] = prod[:, H:]


def _tc_build_entw(entity_emb, weight):
    bm = 1000
    return pl.pallas_call(
        _entw_body,
        grid=(NR, N_ENT // bm),
        in_specs=[
            pl.BlockSpec((bm, D), lambda r, i: (i, 0)),
            pl.BlockSpec((NR, D), lambda r, i: (0, 0)),
        ],
        out_specs=[
            pl.BlockSpec((1, bm, H), lambda r, i: (r, i, 0)),
            pl.BlockSpec((1, bm, H), lambda r, i: (r, i, 0)),
        ],
        out_shape=[
            jax.ShapeDtypeStruct((NR, N_ENT, H), _f32),
            jax.ShapeDtypeStruct((NR, N_ENT, H), _f32),
        ],
    )(entity_emb, weight)


def _softmax_body(x_ref, o_ref):
    x = x_ref[...]
    m = jnp.max(x, axis=1, keepdims=True)
    e = jnp.exp(x - m)
    o_ref[...] = e / jnp.sum(e, axis=1, keepdims=True)


def _tc_softmax(x):
    m, n = x.shape
    bm = 1000
    return pl.pallas_call(
        _softmax_body,
        grid=(m // bm,),
        in_specs=[pl.BlockSpec((bm, n), lambda i: (i, 0))],
        out_specs=pl.BlockSpec((bm, n), lambda i: (i, 0)),
        out_shape=jax.ShapeDtypeStruct((m, n), _f32),
    )(x)


def _entity_gate_body(kg_ref, cnt_ref, agg_ref, wik_ref, bik_ref, win_ref,
                      bin_ref, o_ref):
    pid = pl.program_id(0)
    kg = kg_ref[...]
    c = jnp.maximum(cnt_ref[...][:, :1], 1.0)
    kgd = kg / c
    agg = agg_ref[...]
    ya = lax.dot_general(kgd, wik_ref[...], (((1,), (1,)), ((), ())),
                         preferred_element_type=_f32) + bik_ref[...]
    yb = lax.dot_general(agg, win_ref[...], (((1,), (1,)), ((), ())),
                         preferred_element_type=_f32) + bin_ref[...]
    s = jax.nn.sigmoid(ya + yb)
    blend = s * kgd + (1.0 - s) * agg
    o_ref[...] = jnp.where(pid < N_ITEM // 1000, blend, kgd)


def _tc_entity_gate(kg_sum, cnt16, agg_item, Wik, bik, Win, bin_):
    bm = 1000
    nb_item = N_ITEM // bm
    return pl.pallas_call(
        _entity_gate_body,
        grid=(N_ENT // bm,),
        in_specs=[
            pl.BlockSpec((bm, D), lambda i: (i, 0)),
            pl.BlockSpec((bm, 16), lambda i: (i, 0)),
            pl.BlockSpec((bm, D), lambda i: (jnp.minimum(i, nb_item - 1), 0)),
            pl.BlockSpec((D, D), lambda i: (0, 0)),
            pl.BlockSpec((1, D), lambda i: (0, 0)),
            pl.BlockSpec((D, D), lambda i: (0, 0)),
            pl.BlockSpec((1, D), lambda i: (0, 0)),
        ],
        out_specs=pl.BlockSpec((bm, D), lambda i: (i, 0)),
        out_shape=jax.ShapeDtypeStruct((N_ENT, D), _f32),
    )(kg_sum, cnt16, agg_item, Wik, bik.reshape(1, D), Win, bin_.reshape(1, D))


def _user_gate_body(ui_ref, co_ref, wui_ref, bui_ref, wun_ref, bun_ref, o_ref):
    ui = ui_ref[...]
    co = co_ref[...]
    ya = lax.dot_general(ui, wui_ref[...], (((1,), (1,)), ((), ())),
                         preferred_element_type=_f32) + bui_ref[...]
    yb = lax.dot_general(co, wun_ref[...], (((1,), (1,)), ((), ())),
                         preferred_element_type=_f32) + bun_ref[...]
    g = jax.nn.sigmoid(ya + yb)
    o_ref[...] = g * ui + (1.0 - g) * co


def _tc_user_gate(ui, co, Wui, bui, Wun, bun):
    bm = 1000
    return pl.pallas_call(
        _user_gate_body,
        grid=(N_USER // bm,),
        in_specs=[
            pl.BlockSpec((bm, D), lambda i: (i, 0)),
            pl.BlockSpec((bm, D), lambda i: (i, 0)),
            pl.BlockSpec((D, D), lambda i: (0, 0)),
            pl.BlockSpec((1, D), lambda i: (0, 0)),
            pl.BlockSpec((D, D), lambda i: (0, 0)),
            pl.BlockSpec((1, D), lambda i: (0, 0)),
        ],
        out_specs=pl.BlockSpec((bm, D), lambda i: (i, 0)),
        out_shape=jax.ShapeDtypeStruct((N_USER, D), _f32),
    )(ui, co, Wui, bui.reshape(1, D), Wun, bun.reshape(1, D))


def _reg_body(wa, ba, wik, bik, win, bin_, wui, bui, wun, bun, o_ref):
    total = (jnp.sum(wa[...] ** 2) + jnp.sum(ba[...] ** 2)
             + jnp.sum(wik[...] ** 2) + jnp.sum(bik[...] ** 2)
             + jnp.sum(win[...] ** 2) + jnp.sum(bin_[...] ** 2)
             + jnp.sum(wui[...] ** 2) + jnp.sum(bui[...] ** 2)
             + jnp.sum(wun[...] ** 2) + jnp.sum(bun[...] ** 2))
    o_ref[...] = jnp.reshape(total, (1, 1))


def _tc_reg(Wa, ba, Wik, bik, Win, bin_, Wui, bui, Wun, bun):
    mats = [Wa, Wik, Win, Wui, Wun]
    vecs = [ba, bik, bin_, bui, bun]
    args = []
    in_specs = []
    for m, v in zip(mats, vecs):
        args += [m, v.reshape(1, D)]
        in_specs += [pl.BlockSpec((D, D), lambda: (0, 0)),
                     pl.BlockSpec((1, D), lambda: (0, 0))]
    out = pl.pallas_call(
        _reg_body,
        in_specs=in_specs,
        out_specs=pl.BlockSpec((1, 1), lambda: (0, 0)),
        out_shape=jax.ShapeDtypeStruct((1, 1), _f32),
    )(*args)
    return out.reshape(())


def _seg_sum(vals, idx, n):
    return jax.ops.segment_sum(vals, idx, num_segments=n)


def kernel(entity_emb, user_emb, edge_index, edge_type, weight, iu_row,
           iu_col, iu_val, un_row, un_col, un_val, in_row, in_col, in_val,
           sample_user_item, Wa, ba, Wik, bik, Win, bin_, Wui, bui, Wun, bun):
    head = edge_index[0].astype(jnp.int32)
    tail = edge_index[1].astype(jnp.int32)
    rel = ((edge_type.astype(jnp.int32) + NR - 1) % NR)

    item_emb = entity_emb[:N_ITEM]

    tanhA = _tc_tanh_linear(item_emb, Wa, ba)
    SF = _tc_matmul_nt(user_emb, tanhA)
    entw_lo, entw_hi = _tc_build_entw(entity_emb, weight)

    gidx = rel * N_ENT + tail
    entw = jnp.concatenate(
        [entw_lo.reshape(NR * N_ENT, H), entw_hi.reshape(NR * N_ENT, H)],
        axis=1)
    kg_sum = _seg_sum(entw[gidx], head, N_ENT)
    cnt = _seg_sum(jnp.ones((E,), _f32), head, N_ENT)
    cnt16 = jnp.broadcast_to(cnt[:, None], (N_ENT, 16))

    agg_item = _seg_sum(in_val[:, None] * item_emb[in_col], in_row, N_ITEM)
    user_collab = _seg_sum(un_val[:, None] * user_emb[un_col], un_row, N_USER)

    sc_scores = jnp.take_along_axis(SF, sample_user_item, axis=1)

    p = _tc_softmax(sc_scores)

    srow = jnp.repeat(jnp.arange(N_USER, dtype=jnp.int32), NS)
    scol = sample_user_item.reshape(-1).astype(jnp.int32)
    sval = p.reshape(-1)
    user_item_agg = _seg_sum(sval[:, None] * item_emb[scol], srow, N_USER)

    entity_agg_emb = _tc_entity_gate(kg_sum, cnt16, agg_item, Wik, bik, Win,
                                     bin_)
    user_agg_emb = _tc_user_gate(user_item_agg, user_collab, Wui, bui, Wun,
                                 bun)
    reg = _tc_reg(Wa, ba, Wik, bik, Win, bin_, Wui, bui, Wun, bun)
    return (entity_agg_emb, user_agg_emb, reg)
